# baseline (speedup 1.0000x reference)
import functools

import numpy as np
import jax
import jax.numpy as jnp
from jax import lax
from jax.experimental import pallas as pl
from jax.experimental.pallas import tpu as pltpu
from jax.experimental.pallas import tpu_sc as plsc

N_SAMPLES = 32
BETA = 1.0
SUB = 256
R_SC = 3840
NW = 32

_ROT0 = (13, 15, 26, 6)
_ROT1 = (17, 29, 16, 24)


def _np_threefry2x32(k0, k1, x0, x1):
    x0 = np.uint32(x0)
    x1 = np.uint32(x1)
    ks = [np.uint32(k0), np.uint32(k1),
          np.uint32(np.uint32(k0) ^ np.uint32(k1) ^ np.uint32(0x1BD11BDA))]
    x0 = np.uint32(x0 + ks[0])
    x1 = np.uint32(x1 + ks[1])
    for i in range(5):
        for r in (_ROT0 if i % 2 == 0 else _ROT1):
            x0 = np.uint32(x0 + x1)
            x1 = np.uint32((np.uint32(x1 << np.uint32(r)) |
                            np.uint32(x1 >> np.uint32(32 - r))))
            x1 = np.uint32(x1 ^ x0)
        x0 = np.uint32(x0 + ks[(i + 1) % 3])
        x1 = np.uint32(x1 + ks[(i + 2) % 3] + np.uint32(i + 1))
    return x0, x1


def _sample_key():
    with np.errstate(over="ignore"):
        k0, k1 = _np_threefry2x32(np.uint32(0), np.uint32(42),
                                  np.uint32(0), np.uint32(1))
    return int(k0), int(k1)


_K0, _K1 = _sample_key()
_K2 = (_K0 ^ _K1 ^ 0x1BD11BDA) & 0xFFFFFFFF


def _i32(v):
    return np.int32(np.uint32(v & 0xFFFFFFFF).view(np.int32))


def _rotl(x, r):
    return lax.shift_left(x, np.int32(r)) | lax.shift_right_logical(
        x, np.int32(32 - r))


_KS = (_K0, _K1, _K2)


def _threefry_bits(x1):
    x0 = x1 + _i32(_K0)
    x1 = _rotl(x1, _ROT0[0]) ^ x0
    for r in _ROT0[1:]:
        x0 = x0 + x1
        x1 = _rotl(x1, r) ^ x0
    x0 = x0 + _i32(_KS[1])
    x1 = x1 + _i32(_KS[2] + 1)
    for i in range(1, 5):
        for r in (_ROT0 if i % 2 == 0 else _ROT1):
            x0 = x0 + x1
            x1 = _rotl(x1, r) ^ x0
        x0 = x0 + _i32(_KS[(i + 1) % 3])
        x1 = x1 + _i32(_KS[(i + 2) % 3] + i + 1)
    return x0 ^ x1


def _thresholds(x, w, b2):
    h = lax.dot_general(x, w,
                        dimension_numbers=(((1,), (1,)), ((), ())),
                        preferred_element_type=jnp.float32)
    h = h + b2
    p = jax.nn.sigmoid(2.0 * BETA * h)
    t = p * np.float32(8388608.0)
    tf = t.astype(jnp.int32)
    return tf + (tf.astype(jnp.float32) < t).astype(jnp.int32)


def _kernel_body(x_ref, w_ref, b_ref, o_ref, th_ref, *, blk, n_elem):
    pid = pl.program_id(0)
    th_ref[...] = _thresholds(x_ref[...], w_ref[...], b_ref[...])

    out_f = o_ref.shape[1]
    cols = lax.broadcasted_iota(jnp.int32, (SUB, out_f), 1)
    rows0 = lax.broadcasted_iota(jnp.int32, (SUB, out_f), 0)

    def tile_body(i, carry):
        it = th_ref[pl.ds(i * SUB, SUB), :]
        base = (rows0 + pid * np.int32(blk) + i * np.int32(SUB)) \
            * np.int32(out_f) + cols + _i32(_K1)
        acc = jnp.zeros((SUB, out_f), jnp.int32)
        for s in range(N_SAMPLES):
            m = lax.shift_right_logical(
                _threefry_bits(base + _i32(s * n_elem)), np.int32(9))
            acc = acc + lax.shift_right_arithmetic(m - it, np.int32(31))
        o_ref[pl.ds(i * SUB, SUB), :] = (
            acc.astype(jnp.float32) * np.float32(-1.0 / 16.0)
            - np.float32(1.0))
        return carry

    lax.fori_loop(0, blk // SUB, tile_body, 0)


def _th_body(x_ref, w_ref, b_ref, th_ref):
    th_ref[...] = _thresholds(x_ref[...], w_ref[...], b_ref[...])


def _tc_sample(x, W, b2, n_elem):
    batch, in_f = x.shape
    out_f = W.shape[0]
    blk = next((c for c in (512, 256, SUB) if batch % c == 0), batch)
    body = functools.partial(_kernel_body, blk=blk, n_elem=n_elem)
    return pl.pallas_call(
        body,
        grid=(batch // blk,),
        in_specs=[
            pl.BlockSpec((blk, in_f), lambda i: (i, 0)),
            pl.BlockSpec((out_f, in_f), lambda i: (0, 0)),
            pl.BlockSpec((1, out_f), lambda i: (0, 0)),
        ],
        out_specs=pl.BlockSpec((blk, out_f), lambda i: (i, 0)),
        out_shape=jax.ShapeDtypeStruct((batch, out_f), jnp.float32),
        scratch_shapes=[pltpu.VMEM((blk, out_f), jnp.int32)],
    )(x, W, b2)


def _make_sc_sampler(r_sc, row0, n_elem, out_f):
    words = (r_sc // NW) * out_f
    ncg = words // 16
    mesh = plsc.VectorSubcoreMesh(core_axis_name="c", subcore_axis_name="s")

    @functools.partial(
        pl.kernel, mesh=mesh,
        out_type=jax.ShapeDtypeStruct((r_sc * out_f,), jnp.float32),
        scratch_types=[
            pltpu.VMEM((words,), jnp.int32),
            pltpu.VMEM((words,), jnp.float32),
        ],
    )
    def sc_sample(th_hbm, out_hbm, th_v, out_v):
        wid = lax.axis_index("s") * 2 + lax.axis_index("c")
        base = wid * np.int32(words)
        pltpu.sync_copy(th_hbm.at[pl.ds(base, words)], th_v)
        lane = lax.iota(jnp.int32, 16)

        def cg_body(k, carry):
            it = th_v[pl.ds(k * 16, 16)]
            cnt = lane + (base + k * np.int32(16)
                          + _i32(row0 * out_f + _K1))
            acc = jnp.zeros((16,), jnp.int32)
            for s in range(N_SAMPLES):
                m = lax.shift_right_logical(
                    _threefry_bits(cnt + _i32(s * n_elem)), np.int32(9))
                acc = acc + lax.shift_right_arithmetic(m - it, np.int32(31))
            out_v[pl.ds(k * 16, 16)] = (
                acc.astype(jnp.float32) * np.float32(-1.0 / 16.0)
                - np.float32(1.0))
            return carry

        lax.fori_loop(0, ncg, cg_body, 0)
        pltpu.sync_copy(out_v, out_hbm.at[pl.ds(base, words)])

    return sc_sample


def kernel(x, W, b):
    batch, in_f = x.shape
    out_f = W.shape[0]
    n_elem = batch * out_f
    b2 = jnp.reshape(b, (1, out_f))
    r_sc = R_SC if (batch % 512 == 0 and R_SC % 128 == 0
                    and batch > R_SC and (batch - R_SC) % 256 == 0) else 0
    if not r_sc:
        return _tc_sample(x, W, b2, n_elem)
    r_tc = batch - r_sc
    blk_th = 512 if r_sc % 512 == 0 else 128
    th = pl.pallas_call(
        _th_body,
        grid=(r_sc // blk_th,),
        in_specs=[
            pl.BlockSpec((blk_th, in_f), lambda i: (i, 0)),
            pl.BlockSpec((out_f, in_f), lambda i: (0, 0)),
            pl.BlockSpec((1, out_f), lambda i: (0, 0)),
        ],
        out_specs=pl.BlockSpec((blk_th, out_f), lambda i: (i, 0)),
        out_shape=jax.ShapeDtypeStruct((r_sc, out_f), jnp.int32),
    )(x[r_tc:], W, b2)
    sc_out = _make_sc_sampler(r_sc, r_tc, n_elem, out_f)(
        jnp.reshape(th, (-1,)))
    tc_out = _tc_sample(x[:r_tc], W, b2, n_elem)
    return jnp.concatenate(
        [tc_out, jnp.reshape(sc_out, (r_sc, out_f))], axis=0)


# speedup vs baseline: 1.1962x; 1.1962x over previous
import functools

import numpy as np
import jax
import jax.numpy as jnp
from jax import lax
from jax.experimental import pallas as pl
from jax.experimental.pallas import tpu as pltpu
from jax.experimental.pallas import tpu_sc as plsc

N_SAMPLES = 32
BETA = 1.0
SUB = 256
R_SC = 3584
NW = 32

_ROT0 = (13, 15, 26, 6)
_ROT1 = (17, 29, 16, 24)


def _np_threefry2x32(k0, k1, x0, x1):
    x0 = np.uint32(x0)
    x1 = np.uint32(x1)
    ks = [np.uint32(k0), np.uint32(k1),
          np.uint32(np.uint32(k0) ^ np.uint32(k1) ^ np.uint32(0x1BD11BDA))]
    x0 = np.uint32(x0 + ks[0])
    x1 = np.uint32(x1 + ks[1])
    for i in range(5):
        for r in (_ROT0 if i % 2 == 0 else _ROT1):
            x0 = np.uint32(x0 + x1)
            x1 = np.uint32((np.uint32(x1 << np.uint32(r)) |
                            np.uint32(x1 >> np.uint32(32 - r))))
            x1 = np.uint32(x1 ^ x0)
        x0 = np.uint32(x0 + ks[(i + 1) % 3])
        x1 = np.uint32(x1 + ks[(i + 2) % 3] + np.uint32(i + 1))
    return x0, x1


def _sample_key():
    with np.errstate(over="ignore"):
        k0, k1 = _np_threefry2x32(np.uint32(0), np.uint32(42),
                                  np.uint32(0), np.uint32(1))
    return int(k0), int(k1)


_K0, _K1 = _sample_key()
_K2 = (_K0 ^ _K1 ^ 0x1BD11BDA) & 0xFFFFFFFF


def _i32(v):
    return np.int32(np.uint32(v & 0xFFFFFFFF).view(np.int32))


def _rotl(x, r):
    return lax.shift_left(x, np.int32(r)) | lax.shift_right_logical(
        x, np.int32(32 - r))


_KS = (_K0, _K1, _K2)


def _threefry_bits(x1):
    x0 = x1 + _i32(_K0)
    x1 = _rotl(x1, _ROT0[0]) ^ x0
    for r in _ROT0[1:]:
        x0 = x0 + x1
        x1 = _rotl(x1, r) ^ x0
    x0 = x0 + _i32(_KS[1])
    x1 = x1 + _i32(_KS[2] + 1)
    for i in range(1, 5):
        for r in (_ROT0 if i % 2 == 0 else _ROT1):
            x0 = x0 + x1
            x1 = _rotl(x1, r) ^ x0
        x0 = x0 + _i32(_KS[(i + 1) % 3] + ((1 << 31) if i == 4 else 0))
        x1 = x1 + _i32(_KS[(i + 2) % 3] + i + 1)
    return x0 ^ x1


def _thresholds(x, w, b2):
    h = lax.dot_general(x, w,
                        dimension_numbers=(((1,), (1,)), ((), ())),
                        preferred_element_type=jnp.float32)
    h = h + b2
    p = jax.nn.sigmoid(2.0 * BETA * h)
    t = p * np.float32(8388608.0)
    tf = t.astype(jnp.int32)
    it = tf + (tf.astype(jnp.float32) < t).astype(jnp.int32)
    mn = np.int32(np.uint32(1 << 31).view(np.int32))
    return jnp.where(it == 0, mn,
                     (lax.shift_left(it, np.int32(9)) - np.int32(1)) ^ mn)


def _kernel_body(x_ref, w_ref, b_ref, o_ref, th_ref, *, blk, n_elem):
    pid = pl.program_id(0)
    th_ref[...] = _thresholds(x_ref[...], w_ref[...], b_ref[...])

    out_f = o_ref.shape[1]
    cols = lax.broadcasted_iota(jnp.int32, (SUB, out_f), 1)
    rows0 = lax.broadcasted_iota(jnp.int32, (SUB, out_f), 0)

    def tile_body(i, carry):
        it = th_ref[pl.ds(i * SUB, SUB), :]
        base = (rows0 + pid * np.int32(blk) + i * np.int32(SUB)) \
            * np.int32(out_f) + cols + _i32(_K1)
        acc = jnp.zeros((SUB, out_f), jnp.int32)
        one = jnp.ones((), jnp.int32)
        for s in range(N_SAMPLES):
            bits = _threefry_bits(base + _i32(s * n_elem))
            acc = jnp.where(bits <= it, acc + one, acc)
        o_ref[pl.ds(i * SUB, SUB), :] = (
            acc.astype(jnp.float32) * np.float32(1.0 / 16.0)
            - np.float32(1.0))
        return carry

    lax.fori_loop(0, blk // SUB, tile_body, 0)


def _th_body(x_ref, w_ref, b_ref, th_ref):
    th_ref[...] = _thresholds(x_ref[...], w_ref[...], b_ref[...])


def _tc_sample(x, W, b2, n_elem):
    batch, in_f = x.shape
    out_f = W.shape[0]
    blk = next((c for c in (512, 256, SUB) if batch % c == 0), batch)
    body = functools.partial(_kernel_body, blk=blk, n_elem=n_elem)
    return pl.pallas_call(
        body,
        grid=(batch // blk,),
        in_specs=[
            pl.BlockSpec((blk, in_f), lambda i: (i, 0)),
            pl.BlockSpec((out_f, in_f), lambda i: (0, 0)),
            pl.BlockSpec((1, out_f), lambda i: (0, 0)),
        ],
        out_specs=pl.BlockSpec((blk, out_f), lambda i: (i, 0)),
        out_shape=jax.ShapeDtypeStruct((batch, out_f), jnp.float32),
        scratch_shapes=[pltpu.VMEM((blk, out_f), jnp.int32)],
    )(x, W, b2)


def _make_sc_sampler(r_sc, row0, n_elem, out_f):
    words = (r_sc // NW) * out_f
    ncg = words // 16
    mesh = plsc.VectorSubcoreMesh(core_axis_name="c", subcore_axis_name="s")

    @functools.partial(
        pl.kernel, mesh=mesh,
        out_type=jax.ShapeDtypeStruct((r_sc * out_f,), jnp.float32),
        scratch_types=[
            pltpu.VMEM((words,), jnp.int32),
            pltpu.VMEM((words,), jnp.float32),
        ],
    )
    def sc_sample(th_hbm, out_hbm, th_v, out_v):
        wid = lax.axis_index("s") * 2 + lax.axis_index("c")
        base = wid * np.int32(words)
        pltpu.sync_copy(th_hbm.at[pl.ds(base, words)], th_v)
        lane = lax.iota(jnp.int32, 16)

        def cg_body(k, carry):
            it = th_v[pl.ds(k * 16, 16)]
            cnt = lane + (base + k * np.int32(16)
                          + _i32(row0 * out_f + _K1))
            acc = jnp.zeros((16,), jnp.int32)
            one = jnp.ones((), jnp.int32)
            for s in range(N_SAMPLES):
                bits = _threefry_bits(cnt + _i32(s * n_elem))
                acc = jnp.where(bits <= it, acc + one, acc)
            out_v[pl.ds(k * 16, 16)] = (
                acc.astype(jnp.float32) * np.float32(1.0 / 16.0)
                - np.float32(1.0))
            return carry

        lax.fori_loop(0, ncg, cg_body, 0)
        pltpu.sync_copy(out_v, out_hbm.at[pl.ds(base, words)])

    return sc_sample


def kernel(x, W, b):
    batch, in_f = x.shape
    out_f = W.shape[0]
    n_elem = batch * out_f
    b2 = jnp.reshape(b, (1, out_f))
    r_sc = R_SC if (batch % 512 == 0 and R_SC % 128 == 0
                    and batch > R_SC and (batch - R_SC) % 256 == 0) else 0
    if not r_sc:
        return _tc_sample(x, W, b2, n_elem)
    r_tc = batch - r_sc
    blk_th = 512 if r_sc % 512 == 0 else 128
    th = pl.pallas_call(
        _th_body,
        grid=(r_sc // blk_th,),
        in_specs=[
            pl.BlockSpec((blk_th, in_f), lambda i: (i, 0)),
            pl.BlockSpec((out_f, in_f), lambda i: (0, 0)),
            pl.BlockSpec((1, out_f), lambda i: (0, 0)),
        ],
        out_specs=pl.BlockSpec((blk_th, out_f), lambda i: (i, 0)),
        out_shape=jax.ShapeDtypeStruct((r_sc, out_f), jnp.int32),
    )(x[r_tc:], W, b2)
    sc_out = _make_sc_sampler(r_sc, r_tc, n_elem, out_f)(
        jnp.reshape(th, (-1,)))
    tc_out = _tc_sample(x[:r_tc], W, b2, n_elem)
    return jnp.concatenate(
        [tc_out, jnp.reshape(sc_out, (r_sc, out_f))], axis=0)


# speedup vs baseline: 1.2424x; 1.0387x over previous
import functools

import numpy as np
import jax
import jax.numpy as jnp
from jax import lax
from jax.experimental import pallas as pl
from jax.experimental.pallas import tpu as pltpu
from jax.experimental.pallas import tpu_sc as plsc

N_SAMPLES = 32
BETA = 1.0
SUB = 256
R_SC = 4096
NW = 32

_ROT0 = (13, 15, 26, 6)
_ROT1 = (17, 29, 16, 24)


def _np_threefry2x32(k0, k1, x0, x1):
    x0 = np.uint32(x0)
    x1 = np.uint32(x1)
    ks = [np.uint32(k0), np.uint32(k1),
          np.uint32(np.uint32(k0) ^ np.uint32(k1) ^ np.uint32(0x1BD11BDA))]
    x0 = np.uint32(x0 + ks[0])
    x1 = np.uint32(x1 + ks[1])
    for i in range(5):
        for r in (_ROT0 if i % 2 == 0 else _ROT1):
            x0 = np.uint32(x0 + x1)
            x1 = np.uint32((np.uint32(x1 << np.uint32(r)) |
                            np.uint32(x1 >> np.uint32(32 - r))))
            x1 = np.uint32(x1 ^ x0)
        x0 = np.uint32(x0 + ks[(i + 1) % 3])
        x1 = np.uint32(x1 + ks[(i + 2) % 3] + np.uint32(i + 1))
    return x0, x1


def _sample_key():
    with np.errstate(over="ignore"):
        k0, k1 = _np_threefry2x32(np.uint32(0), np.uint32(42),
                                  np.uint32(0), np.uint32(1))
    return int(k0), int(k1)


_K0, _K1 = _sample_key()
_K2 = (_K0 ^ _K1 ^ 0x1BD11BDA) & 0xFFFFFFFF


def _i32(v):
    return np.int32(np.uint32(v & 0xFFFFFFFF).view(np.int32))


def _rotl(x, r):
    return lax.shift_left(x, np.int32(r)) | lax.shift_right_logical(
        x, np.int32(32 - r))


_KS = (_K0, _K1, _K2)


def _threefry_bits(x1):
    x0 = x1 + _i32(_K0)
    x1 = _rotl(x1, _ROT0[0]) ^ x0
    for r in _ROT0[1:]:
        x0 = x0 + x1
        x1 = _rotl(x1, r) ^ x0
    x0 = x0 + _i32(_KS[1])
    x1 = x1 + _i32(_KS[2] + 1)
    for i in range(1, 5):
        for r in (_ROT0 if i % 2 == 0 else _ROT1):
            x0 = x0 + x1
            x1 = _rotl(x1, r) ^ x0
        x0 = x0 + _i32(_KS[(i + 1) % 3] + ((1 << 31) if i == 4 else 0))
        x1 = x1 + _i32(_KS[(i + 2) % 3] + i + 1)
    return x0 ^ x1


def _thresholds(x, w, b2):
    h = lax.dot_general(x, w,
                        dimension_numbers=(((1,), (1,)), ((), ())),
                        preferred_element_type=jnp.float32)
    h = h + b2
    p = jax.nn.sigmoid(2.0 * BETA * h)
    t = p * np.float32(8388608.0)
    tf = t.astype(jnp.int32)
    it = tf + (tf.astype(jnp.float32) < t).astype(jnp.int32)
    mn = np.int32(np.uint32(1 << 31).view(np.int32))
    return jnp.where(it == 0, mn,
                     (lax.shift_left(it, np.int32(9)) - np.int32(1)) ^ mn)


def _kernel_body(x_ref, w_ref, b_ref, o_ref, th_ref, *, blk, n_elem):
    pid = pl.program_id(0)
    th_ref[...] = _thresholds(x_ref[...], w_ref[...], b_ref[...])

    out_f = o_ref.shape[1]
    cols = lax.broadcasted_iota(jnp.int32, (SUB, out_f), 1)
    rows0 = lax.broadcasted_iota(jnp.int32, (SUB, out_f), 0)

    def tile_body(i, carry):
        it = th_ref[pl.ds(i * SUB, SUB), :]
        base = (rows0 + pid * np.int32(blk) + i * np.int32(SUB)) \
            * np.int32(out_f) + cols + _i32(_K1)
        acc = jnp.zeros((SUB, out_f), jnp.int32)
        one = jnp.ones((), jnp.int32)
        for s in range(N_SAMPLES):
            bits = _threefry_bits(base + _i32(s * n_elem))
            acc = jnp.where(bits <= it, acc + one, acc)
        o_ref[pl.ds(i * SUB, SUB), :] = (
            acc.astype(jnp.float32) * np.float32(1.0 / 16.0)
            - np.float32(1.0))
        return carry

    lax.fori_loop(0, blk // SUB, tile_body, 0)


def _th_body(x_ref, w_ref, b_ref, th_ref):
    th_ref[...] = _thresholds(x_ref[...], w_ref[...], b_ref[...])


def _tc_sample(x, W, b2, n_elem):
    batch, in_f = x.shape
    out_f = W.shape[0]
    blk = next((c for c in (512, 256, SUB) if batch % c == 0), batch)
    body = functools.partial(_kernel_body, blk=blk, n_elem=n_elem)
    return pl.pallas_call(
        body,
        grid=(batch // blk,),
        in_specs=[
            pl.BlockSpec((blk, in_f), lambda i: (i, 0)),
            pl.BlockSpec((out_f, in_f), lambda i: (0, 0)),
            pl.BlockSpec((1, out_f), lambda i: (0, 0)),
        ],
        out_specs=pl.BlockSpec((blk, out_f), lambda i: (i, 0)),
        out_shape=jax.ShapeDtypeStruct((batch, out_f), jnp.float32),
        scratch_shapes=[pltpu.VMEM((blk, out_f), jnp.int32)],
    )(x, W, b2)


def _make_sc_sampler(r_sc, row0, n_elem, out_f):
    words = (r_sc // NW) * out_f
    ncg = words // 16
    mesh = plsc.VectorSubcoreMesh(core_axis_name="c", subcore_axis_name="s")

    @functools.partial(
        pl.kernel, mesh=mesh,
        out_type=jax.ShapeDtypeStruct((r_sc * out_f,), jnp.float32),
        scratch_types=[
            pltpu.VMEM((words,), jnp.int32),
            pltpu.VMEM((words,), jnp.float32),
        ],
    )
    def sc_sample(th_hbm, out_hbm, th_v, out_v):
        wid = lax.axis_index("s") * 2 + lax.axis_index("c")
        base = wid * np.int32(words)
        pltpu.sync_copy(th_hbm.at[pl.ds(base, words)], th_v)
        lane = lax.iota(jnp.int32, 16)

        def cg_body(k, carry):
            it = th_v[pl.ds(k * 16, 16)]
            cnt = lane + (base + k * np.int32(16)
                          + _i32(row0 * out_f + _K1))
            acc = jnp.zeros((16,), jnp.int32)
            one = jnp.ones((), jnp.int32)
            for s in range(N_SAMPLES):
                bits = _threefry_bits(cnt + _i32(s * n_elem))
                acc = jnp.where(bits <= it, acc + one, acc)
            out_v[pl.ds(k * 16, 16)] = (
                acc.astype(jnp.float32) * np.float32(1.0 / 16.0)
                - np.float32(1.0))
            return carry

        lax.fori_loop(0, ncg, cg_body, 0)
        pltpu.sync_copy(out_v, out_hbm.at[pl.ds(base, words)])

    return sc_sample


def kernel(x, W, b):
    batch, in_f = x.shape
    out_f = W.shape[0]
    n_elem = batch * out_f
    b2 = jnp.reshape(b, (1, out_f))
    r_sc = R_SC if (batch % 512 == 0 and R_SC % 128 == 0
                    and batch > R_SC and (batch - R_SC) % 256 == 0) else 0
    if not r_sc:
        return _tc_sample(x, W, b2, n_elem)
    r_tc = batch - r_sc
    blk_th = 512 if r_sc % 512 == 0 else 128
    th = pl.pallas_call(
        _th_body,
        grid=(r_sc // blk_th,),
        in_specs=[
            pl.BlockSpec((blk_th, in_f), lambda i: (i, 0)),
            pl.BlockSpec((out_f, in_f), lambda i: (0, 0)),
            pl.BlockSpec((1, out_f), lambda i: (0, 0)),
        ],
        out_specs=pl.BlockSpec((blk_th, out_f), lambda i: (i, 0)),
        out_shape=jax.ShapeDtypeStruct((r_sc, out_f), jnp.int32),
    )(x[r_tc:], W, b2)
    sc_out = _make_sc_sampler(r_sc, r_tc, n_elem, out_f)(
        jnp.reshape(th, (-1,)))
    tc_out = _tc_sample(x[:r_tc], W, b2, n_elem)
    return jnp.concatenate(
        [tc_out, jnp.reshape(sc_out, (r_sc, out_f))], axis=0)
